# baseline (speedup 1.0000x reference)
import functools

import jax
import jax.numpy as jnp
from jax import lax
from jax.experimental import pallas as pl
from jax.experimental.pallas import tpu as pltpu
from jax.experimental.pallas import tpu_sc as plsc

N, E, D = 10000, 320000, 128
H = D // 2
NP = 10240
NC, NS, L = 2, 16, 16
CHUNK = 128
NCH = 160
NPH = 2
CPH = NCH // NPH
EPT = NCH * CHUNK
EPAD = NS * EPT
RPT = NP // NS
RPT_LAST = N - (NS - 1) * RPT
ZROWS = 128


def _sc_scatter_add(xbf, srcs4, dsts4):
    mesh = plsc.VectorSubcoreMesh(
        core_axis_name="c", subcore_axis_name="s", num_cores=NC, num_subcores=NS
    )

    @functools.partial(
        pl.kernel,
        out_type=jax.ShapeDtypeStruct((2, NP, D), jnp.bfloat16),
        mesh=mesh,
        scratch_types=[
            pltpu.VMEM((CPH, CHUNK), jnp.int32),
            pltpu.VMEM((CPH, CHUNK), jnp.int32),
            pltpu.VMEM((CHUNK, H), jnp.bfloat16),
            pltpu.VMEM((CHUNK, H), jnp.bfloat16),
            pltpu.VMEM_SHARED((NP, H), jnp.bfloat16),
            pltpu.VMEM_SHARED((NP, H), jnp.bfloat16),
            pltpu.VMEM_SHARED((NP, H), jnp.bfloat16),
            pltpu.SemaphoreType.DMA,
            pltpu.SemaphoreType.DMA,
            pltpu.SemaphoreType.DMA,
            pltpu.SemaphoreType.DMA,
            pltpu.SemaphoreType.DMA,
        ],
        compiler_params=pltpu.CompilerParams(use_tc_tiling_on_sc=False),
    )
    def k(xbf_hbm, srcs_hbm, dsts_hbm, agg_hbm,
          sb, db, rows0, rows1, xspm, agg0, agg1,
          isem, gsem0, gsem1, ssem0, ssem1):
        cid = lax.axis_index("c")
        sid = lax.axis_index("s")
        r0 = sid * RPT
        c0 = cid * H

        pltpu.async_copy(srcs_hbm.at[sid, 0], sb, isem)
        pltpu.async_copy(dsts_hbm.at[sid, 0], db, isem)

        @pl.when(sid < NS - 1)
        def _():
            pltpu.sync_copy(xbf_hbm.at[pl.ds(r0, RPT), pl.ds(c0, H)],
                            xspm.at[pl.ds(r0, RPT)])

        @pl.when(sid == NS - 1)
        def _():
            pltpu.sync_copy(xbf_hbm.at[pl.ds(r0, RPT_LAST), pl.ds(c0, H)],
                            xspm.at[pl.ds(r0, RPT_LAST)])

        def zrow(i, _):
            def zcol(c, _):
                rows0[i, pl.ds(c * 2 * L, 2 * L)] = jnp.zeros((2 * L,), jnp.bfloat16)
                return 0
            return lax.fori_loop(0, H // (2 * L), zcol, 0)
        lax.fori_loop(0, ZROWS, zrow, 0)

        for agg in (agg0, agg1):
            for j in range(RPT // ZROWS):
                pltpu.sync_copy(rows0, agg.at[pl.ds(r0 + j * ZROWS, ZROWS)])
        plsc.subcore_barrier()

        def g_wait(rows, gsem):
            pltpu.make_async_copy(xspm.at[sb.at[0]], rows, gsem).wait()

        def s_wait(rows, ssem, agg):
            pltpu.make_async_copy(rows, agg.at[db.at[0]], ssem).wait()

        for ph in range(NPH):
            agg = agg0 if ph == 0 else agg1
            pltpu.make_async_copy(srcs_hbm.at[sid, ph], sb, isem).wait()
            pltpu.make_async_copy(dsts_hbm.at[sid, ph], db, isem).wait()

            pltpu.async_copy(xspm.at[sb.at[0]], rows0, gsem0)

            def body(g, _):
                i0 = 2 * g
                g_wait(rows0, gsem0)

                @pl.when(g > 0)
                def _():
                    s_wait(rows1, ssem1, agg)

                pltpu.async_copy(xspm.at[sb.at[i0 + 1]], rows1, gsem1)
                pltpu.async_copy(rows0, agg.at[db.at[i0]], ssem0, add=True)

                g_wait(rows1, gsem1)
                s_wait(rows0, ssem0, agg)

                @pl.when(g < CPH // 2 - 1)
                def _():
                    pltpu.async_copy(xspm.at[sb.at[i0 + 2]], rows0, gsem0)

                pltpu.async_copy(rows1, agg.at[db.at[i0 + 1]], ssem1, add=True)
                return 0

            lax.fori_loop(0, CPH // 2, body, 0)
            s_wait(rows1, ssem1, agg)
            if ph + 1 < NPH:
                pltpu.async_copy(srcs_hbm.at[sid, ph + 1], sb, isem)
                pltpu.async_copy(dsts_hbm.at[sid, ph + 1], db, isem)

        plsc.subcore_barrier()

        pltpu.sync_copy(agg0.at[pl.ds(r0, RPT)],
                        agg_hbm.at[0, pl.ds(r0, RPT), pl.ds(c0, H)])
        pltpu.sync_copy(agg1.at[pl.ds(r0, RPT)],
                        agg_hbm.at[1, pl.ds(r0, RPT), pl.ds(c0, H)])

    return k(xbf, srcs4, dsts4)


_BLK = 400


def _mlp_body(a0_ref, a1_ref, x_ref, w1_ref, b1_ref, w2_ref, b2_ref, o_ref):
    s = (a0_ref[0].astype(jnp.float32) + a1_ref[0].astype(jnp.float32)
         + x_ref[...])
    h = jnp.maximum(
        jnp.dot(s, w1_ref[...], preferred_element_type=jnp.float32) + b1_ref[...], 0.0
    )
    o_ref[...] = jnp.dot(h, w2_ref[...], preferred_element_type=jnp.float32) + b2_ref[...]


def _mlp(agg, x, W1, b1, W2, b2):
    return pl.pallas_call(
        _mlp_body,
        grid=(N // _BLK,),
        in_specs=[
            pl.BlockSpec((1, _BLK, D), lambda i: (0, i, 0)),
            pl.BlockSpec((1, _BLK, D), lambda i: (1, i, 0)),
            pl.BlockSpec((_BLK, D), lambda i: (i, 0)),
            pl.BlockSpec((D, D), lambda i: (0, 0)),
            pl.BlockSpec((1, D), lambda i: (0, 0)),
            pl.BlockSpec((D, D), lambda i: (0, 0)),
            pl.BlockSpec((1, D), lambda i: (0, 0)),
        ],
        out_specs=pl.BlockSpec((_BLK, D), lambda i: (i, 0)),
        out_shape=jax.ShapeDtypeStruct((N, D), jnp.float32),
    )(agg, agg, x, W1, b1.reshape(1, D), W2, b2.reshape(1, D))


@jax.jit
def kernel(x, edge_index, W1, b1, W2, b2):
    src = edge_index[0]
    dst = edge_index[1]
    pad = EPAD - E
    src_p = jnp.concatenate([src, jnp.zeros((pad,), jnp.int32)])
    dst_p = jnp.concatenate([dst, jnp.full((pad,), N, jnp.int32)])
    srcs4 = src_p.reshape(NS, NPH, CPH, CHUNK)
    dsts4 = dst_p.reshape(NS, NPH, CPH, CHUNK)
    agg = _sc_scatter_add(x.astype(jnp.bfloat16), srcs4, dsts4)
    return _mlp(agg, x, W1, b1, W2, b2)


# speedup vs baseline: 1.0143x; 1.0143x over previous
import functools

import jax
import jax.numpy as jnp
from jax import lax
from jax.experimental import pallas as pl
from jax.experimental.pallas import tpu as pltpu
from jax.experimental.pallas import tpu_sc as plsc

N, E, D = 10000, 320000, 128
H = D // 2
NP = 10240
NC, NS, L = 2, 16, 16
CHUNK = 128
NCH = 160
NPH = 2
CPH = NCH // NPH
EPT = NCH * CHUNK
EPAD = NS * EPT
RPT = NP // NS
RPT_LAST = N - (NS - 1) * RPT
ZROWS = 128


def _sc_scatter_add(xbf, srcs4, dsts4):
    mesh = plsc.VectorSubcoreMesh(
        core_axis_name="c", subcore_axis_name="s", num_cores=NC, num_subcores=NS
    )

    @functools.partial(
        pl.kernel,
        out_type=jax.ShapeDtypeStruct((2, NP, D), jnp.bfloat16),
        mesh=mesh,
        scratch_types=[
            pltpu.VMEM((CPH, CHUNK), jnp.int32),
            pltpu.VMEM((CPH, CHUNK), jnp.int32),
            pltpu.VMEM((CHUNK, H), jnp.bfloat16),
            pltpu.VMEM((CHUNK, H), jnp.bfloat16),
            pltpu.VMEM_SHARED((NP, H), jnp.bfloat16),
            pltpu.VMEM_SHARED((NP, H), jnp.bfloat16),
            pltpu.VMEM_SHARED((NP, H), jnp.bfloat16),
            pltpu.SemaphoreType.DMA,
            pltpu.SemaphoreType.DMA,
            pltpu.SemaphoreType.DMA,
            pltpu.SemaphoreType.DMA,
            pltpu.SemaphoreType.DMA,
        ],
        compiler_params=pltpu.CompilerParams(use_tc_tiling_on_sc=False),
    )
    def k(xbf_hbm, srcs_hbm, dsts_hbm, agg_hbm,
          sb, db, rows0, rows1, xspm, agg0, agg1,
          isem, gsem0, gsem1, ssem0, ssem1):
        cid = lax.axis_index("c")
        sid = lax.axis_index("s")
        r0 = sid * RPT
        c0 = cid * H

        pltpu.async_copy(srcs_hbm.at[sid, 0], sb, isem)
        pltpu.async_copy(dsts_hbm.at[sid, 0], db, isem)

        @pl.when(sid < NS - 1)
        def _():
            pltpu.async_copy(xbf_hbm.at[pl.ds(r0, RPT), pl.ds(c0, H)],
                             xspm.at[pl.ds(r0, RPT)], gsem0)

        @pl.when(sid == NS - 1)
        def _():
            pltpu.async_copy(xbf_hbm.at[pl.ds(r0, RPT_LAST), pl.ds(c0, H)],
                             xspm.at[pl.ds(r0, RPT_LAST)], gsem0)

        def zrow(i, _):
            def zcol(c, _):
                rows0[i, pl.ds(c * 2 * L, 2 * L)] = jnp.zeros((2 * L,), jnp.bfloat16)
                return 0
            return lax.fori_loop(0, H // (2 * L), zcol, 0)
        lax.fori_loop(0, ZROWS, zrow, 0)

        for agg in (agg0, agg1):
            for j in range(RPT // ZROWS):
                pltpu.async_copy(rows0, agg.at[pl.ds(r0 + j * ZROWS, ZROWS)], ssem0)
        for agg in (agg0, agg1):
            for j in range(RPT // ZROWS):
                pltpu.make_async_copy(rows0, agg.at[pl.ds(r0 + j * ZROWS, ZROWS)],
                                      ssem0).wait()

        @pl.when(sid < NS - 1)
        def _():
            pltpu.make_async_copy(xbf_hbm.at[pl.ds(r0, RPT), pl.ds(c0, H)],
                                  xspm.at[pl.ds(r0, RPT)], gsem0).wait()

        @pl.when(sid == NS - 1)
        def _():
            pltpu.make_async_copy(xbf_hbm.at[pl.ds(r0, RPT_LAST), pl.ds(c0, H)],
                                  xspm.at[pl.ds(r0, RPT_LAST)], gsem0).wait()
        plsc.subcore_barrier()

        def g_wait(rows, gsem):
            pltpu.make_async_copy(xspm.at[sb.at[0]], rows, gsem).wait()

        def s_wait(rows, ssem, agg):
            pltpu.make_async_copy(rows, agg.at[db.at[0]], ssem).wait()

        for ph in range(NPH):
            agg = agg0 if ph == 0 else agg1
            pltpu.make_async_copy(srcs_hbm.at[sid, ph], sb, isem).wait()
            pltpu.make_async_copy(dsts_hbm.at[sid, ph], db, isem).wait()

            pltpu.async_copy(xspm.at[sb.at[0]], rows0, gsem0)

            def body(g, _):
                i0 = 2 * g
                g_wait(rows0, gsem0)

                @pl.when(g > 0)
                def _():
                    s_wait(rows1, ssem1, agg)

                pltpu.async_copy(xspm.at[sb.at[i0 + 1]], rows1, gsem1)
                pltpu.async_copy(rows0, agg.at[db.at[i0]], ssem0, add=True)

                g_wait(rows1, gsem1)
                s_wait(rows0, ssem0, agg)

                @pl.when(g < CPH // 2 - 1)
                def _():
                    pltpu.async_copy(xspm.at[sb.at[i0 + 2]], rows0, gsem0)

                pltpu.async_copy(rows1, agg.at[db.at[i0 + 1]], ssem1, add=True)
                return 0

            lax.fori_loop(0, CPH // 2, body, 0)
            s_wait(rows1, ssem1, agg)
            if ph + 1 < NPH:
                pltpu.async_copy(srcs_hbm.at[sid, ph + 1], sb, isem)
                pltpu.async_copy(dsts_hbm.at[sid, ph + 1], db, isem)

        plsc.subcore_barrier()

        pltpu.async_copy(agg0.at[pl.ds(r0, RPT)],
                         agg_hbm.at[0, pl.ds(r0, RPT), pl.ds(c0, H)], gsem0)
        pltpu.async_copy(agg1.at[pl.ds(r0, RPT)],
                         agg_hbm.at[1, pl.ds(r0, RPT), pl.ds(c0, H)], gsem1)
        pltpu.make_async_copy(agg0.at[pl.ds(r0, RPT)],
                              agg_hbm.at[0, pl.ds(r0, RPT), pl.ds(c0, H)],
                              gsem0).wait()
        pltpu.make_async_copy(agg1.at[pl.ds(r0, RPT)],
                              agg_hbm.at[1, pl.ds(r0, RPT), pl.ds(c0, H)],
                              gsem1).wait()

    return k(xbf, srcs4, dsts4)


_BLK = 400


def _mlp_body(a0_ref, a1_ref, x_ref, w1_ref, b1_ref, w2_ref, b2_ref, o_ref):
    s = (a0_ref[0].astype(jnp.float32) + a1_ref[0].astype(jnp.float32)
         + x_ref[...])
    h = jnp.maximum(
        jnp.dot(s, w1_ref[...], preferred_element_type=jnp.float32) + b1_ref[...], 0.0
    )
    o_ref[...] = jnp.dot(h, w2_ref[...], preferred_element_type=jnp.float32) + b2_ref[...]


def _mlp(agg, x, W1, b1, W2, b2):
    return pl.pallas_call(
        _mlp_body,
        grid=(N // _BLK,),
        in_specs=[
            pl.BlockSpec((1, _BLK, D), lambda i: (0, i, 0)),
            pl.BlockSpec((1, _BLK, D), lambda i: (1, i, 0)),
            pl.BlockSpec((_BLK, D), lambda i: (i, 0)),
            pl.BlockSpec((D, D), lambda i: (0, 0)),
            pl.BlockSpec((1, D), lambda i: (0, 0)),
            pl.BlockSpec((D, D), lambda i: (0, 0)),
            pl.BlockSpec((1, D), lambda i: (0, 0)),
        ],
        out_specs=pl.BlockSpec((_BLK, D), lambda i: (i, 0)),
        out_shape=jax.ShapeDtypeStruct((N, D), jnp.float32),
    )(agg, agg, x, W1, b1.reshape(1, D), W2, b2.reshape(1, D))


@jax.jit
def kernel(x, edge_index, W1, b1, W2, b2):
    src = edge_index[0]
    dst = edge_index[1]
    pad = EPAD - E
    src_p = jnp.concatenate([src, jnp.zeros((pad,), jnp.int32)])
    dst_p = jnp.concatenate([dst, jnp.full((pad,), N, jnp.int32)])
    srcs4 = src_p.reshape(NS, NPH, CPH, CHUNK)
    dsts4 = dst_p.reshape(NS, NPH, CPH, CHUNK)
    agg = _sc_scatter_add(x.astype(jnp.bfloat16), srcs4, dsts4)
    return _mlp(agg, x, W1, b1, W2, b2)


# speedup vs baseline: 1.1439x; 1.1278x over previous
import functools

import jax
import jax.numpy as jnp
from jax import lax
from jax.experimental import pallas as pl
from jax.experimental.pallas import tpu as pltpu
from jax.experimental.pallas import tpu_sc as plsc

N, E, D = 10000, 320000, 128
H = D // 2
NP = 10240
NC, NS, L = 2, 16, 16
CHUNK = 128
NCHT = E // CHUNK
CBASE = NCHT // NS
XTILES = NCHT - CBASE * NS
NPH = 2
CPH = CBASE // NPH
RPT = NP // NS
RPT_LAST = N - (NS - 1) * RPT
ZROWS = 128


def _sc_scatter_add(xbf, e3):
    mesh = plsc.VectorSubcoreMesh(
        core_axis_name="c", subcore_axis_name="s", num_cores=NC, num_subcores=NS
    )

    @functools.partial(
        pl.kernel,
        out_type=jax.ShapeDtypeStruct((2, NP, D), jnp.bfloat16),
        mesh=mesh,
        scratch_types=[
            pltpu.VMEM((CPH, CHUNK), jnp.int32),
            pltpu.VMEM((CPH, CHUNK), jnp.int32),
            pltpu.VMEM((CHUNK, H), jnp.bfloat16),
            pltpu.VMEM((CHUNK, H), jnp.bfloat16),
            pltpu.VMEM_SHARED((NP, H), jnp.bfloat16),
            pltpu.VMEM_SHARED((NP, H), jnp.bfloat16),
            pltpu.VMEM_SHARED((NP, H), jnp.bfloat16),
            pltpu.SemaphoreType.DMA,
            pltpu.SemaphoreType.DMA,
            pltpu.SemaphoreType.DMA,
            pltpu.SemaphoreType.DMA,
            pltpu.SemaphoreType.DMA,
        ],
        compiler_params=pltpu.CompilerParams(use_tc_tiling_on_sc=False),
    )
    def k(xbf_hbm, e_hbm, agg_hbm,
          sb, db, rows0, rows1, xspm, agg0, agg1,
          isem, gsem0, gsem1, ssem0, ssem1):
        cid = lax.axis_index("c")
        sid = lax.axis_index("s")
        r0 = sid * RPT
        c0 = cid * H
        base = sid * CBASE + jnp.minimum(sid, XTILES)

        pltpu.async_copy(e_hbm.at[0, pl.ds(base, CPH)], sb, isem)
        pltpu.async_copy(e_hbm.at[1, pl.ds(base, CPH)], db, isem)

        @pl.when(sid < NS - 1)
        def _():
            pltpu.async_copy(xbf_hbm.at[pl.ds(r0, RPT), pl.ds(c0, H)],
                             xspm.at[pl.ds(r0, RPT)], gsem0)

        @pl.when(sid == NS - 1)
        def _():
            pltpu.async_copy(xbf_hbm.at[pl.ds(r0, RPT_LAST), pl.ds(c0, H)],
                             xspm.at[pl.ds(r0, RPT_LAST)], gsem0)

        def zrow(i, _):
            def zcol(c, _):
                rows0[i, pl.ds(c * 2 * L, 2 * L)] = jnp.zeros((2 * L,), jnp.bfloat16)
                return 0
            return lax.fori_loop(0, H // (2 * L), zcol, 0)
        lax.fori_loop(0, ZROWS, zrow, 0)

        for agg in (agg0, agg1):
            for j in range(RPT // ZROWS):
                pltpu.async_copy(rows0, agg.at[pl.ds(r0 + j * ZROWS, ZROWS)], ssem0)
        for agg in (agg0, agg1):
            for j in range(RPT // ZROWS):
                pltpu.make_async_copy(rows0, agg.at[pl.ds(r0 + j * ZROWS, ZROWS)],
                                      ssem0).wait()

        @pl.when(sid < NS - 1)
        def _():
            pltpu.make_async_copy(xbf_hbm.at[pl.ds(r0, RPT), pl.ds(c0, H)],
                                  xspm.at[pl.ds(r0, RPT)], gsem0).wait()

        @pl.when(sid == NS - 1)
        def _():
            pltpu.make_async_copy(xbf_hbm.at[pl.ds(r0, RPT_LAST), pl.ds(c0, H)],
                                  xspm.at[pl.ds(r0, RPT_LAST)], gsem0).wait()
        plsc.subcore_barrier()

        def g_wait(rows, gsem):
            pltpu.make_async_copy(xspm.at[sb.at[0]], rows, gsem).wait()

        def s_wait(rows, ssem, agg):
            pltpu.make_async_copy(rows, agg.at[db.at[0]], ssem).wait()

        for ph in range(NPH):
            agg = agg0 if ph == 0 else agg1
            poff = base + ph * CPH
            pltpu.make_async_copy(e_hbm.at[0, pl.ds(poff, CPH)], sb, isem).wait()
            pltpu.make_async_copy(e_hbm.at[1, pl.ds(poff, CPH)], db, isem).wait()

            pltpu.async_copy(xspm.at[sb.at[0]], rows0, gsem0)

            def body(g, _):
                i0 = 2 * g
                g_wait(rows0, gsem0)

                @pl.when(g > 0)
                def _():
                    s_wait(rows1, ssem1, agg)

                pltpu.async_copy(xspm.at[sb.at[i0 + 1]], rows1, gsem1)
                pltpu.async_copy(rows0, agg.at[db.at[i0]], ssem0, add=True)

                g_wait(rows1, gsem1)
                s_wait(rows0, ssem0, agg)

                @pl.when(g < CPH // 2 - 1)
                def _():
                    pltpu.async_copy(xspm.at[sb.at[i0 + 2]], rows0, gsem0)

                pltpu.async_copy(rows1, agg.at[db.at[i0 + 1]], ssem1, add=True)
                return 0

            lax.fori_loop(0, CPH // 2, body, 0)
            s_wait(rows1, ssem1, agg)
            if ph + 1 < NPH:
                noff = base + (ph + 1) * CPH
                pltpu.async_copy(e_hbm.at[0, pl.ds(noff, CPH)], sb, isem)
                pltpu.async_copy(e_hbm.at[1, pl.ds(noff, CPH)], db, isem)

        @pl.when(sid < XTILES)
        def _():
            xoff = base + CBASE
            pltpu.sync_copy(e_hbm.at[0, pl.ds(xoff, 1)], sb.at[pl.ds(0, 1)])
            pltpu.sync_copy(e_hbm.at[1, pl.ds(xoff, 1)], db.at[pl.ds(0, 1)])
            pltpu.async_copy(xspm.at[sb.at[0]], rows0, gsem0)
            g_wait(rows0, gsem0)
            pltpu.async_copy(rows0, agg1.at[db.at[0]], ssem0, add=True)
            s_wait(rows0, ssem0, agg1)

        plsc.subcore_barrier()

        pltpu.async_copy(agg0.at[pl.ds(r0, RPT)],
                         agg_hbm.at[0, pl.ds(r0, RPT), pl.ds(c0, H)], gsem0)
        pltpu.async_copy(agg1.at[pl.ds(r0, RPT)],
                         agg_hbm.at[1, pl.ds(r0, RPT), pl.ds(c0, H)], gsem1)
        pltpu.make_async_copy(agg0.at[pl.ds(r0, RPT)],
                              agg_hbm.at[0, pl.ds(r0, RPT), pl.ds(c0, H)],
                              gsem0).wait()
        pltpu.make_async_copy(agg1.at[pl.ds(r0, RPT)],
                              agg_hbm.at[1, pl.ds(r0, RPT), pl.ds(c0, H)],
                              gsem1).wait()

    return k(xbf, e3)


_BLK = 1000


def _mlp_body(a0_ref, a1_ref, x_ref, w1_ref, b1_ref, w2_ref, b2_ref, o_ref):
    s = (a0_ref[0].astype(jnp.float32) + a1_ref[0].astype(jnp.float32)
         + x_ref[...])
    h = jnp.maximum(
        jnp.dot(s, w1_ref[...], preferred_element_type=jnp.float32) + b1_ref[...], 0.0
    )
    o_ref[...] = jnp.dot(h, w2_ref[...], preferred_element_type=jnp.float32) + b2_ref[...]


def _mlp(agg, x, W1, b1, W2, b2):
    return pl.pallas_call(
        _mlp_body,
        grid=(N // _BLK,),
        in_specs=[
            pl.BlockSpec((1, _BLK, D), lambda i: (0, i, 0)),
            pl.BlockSpec((1, _BLK, D), lambda i: (1, i, 0)),
            pl.BlockSpec((_BLK, D), lambda i: (i, 0)),
            pl.BlockSpec((D, D), lambda i: (0, 0)),
            pl.BlockSpec((1, D), lambda i: (0, 0)),
            pl.BlockSpec((D, D), lambda i: (0, 0)),
            pl.BlockSpec((1, D), lambda i: (0, 0)),
        ],
        out_specs=pl.BlockSpec((_BLK, D), lambda i: (i, 0)),
        out_shape=jax.ShapeDtypeStruct((N, D), jnp.float32),
    )(agg, agg, x, W1, b1.reshape(1, D), W2, b2.reshape(1, D))


@jax.jit
def kernel(x, edge_index, W1, b1, W2, b2):
    e3 = edge_index.reshape(2, NCHT, CHUNK)
    agg = _sc_scatter_add(x.astype(jnp.bfloat16), e3)
    return _mlp(agg, x, W1, b1, W2, b2)
